# baseline (speedup 1.0000x reference)
import functools

import jax
import jax.numpy as jnp
from jax import lax
from jax.experimental import pallas as pl
from jax.experimental.pallas import tpu as pltpu
from jax.experimental.pallas import tpu_sc as plsc

B = 4096
L = 50
D = 64
LP = 56
NW = 32
RPW = B // NW
NBUF = 2
CR = 8
NCH = RPW // CR


def _body(idx_hbm, val_hbm, emb_hbm, out_hbm,
          idx_slab, val_slab, out_v, buf0, buf1, sem0, sem1):
    w = lax.axis_index("s") * 2 + lax.axis_index("c")
    base = w * RPW

    pltpu.sync_copy(idx_hbm.at[pl.ds(base * LP, RPW * LP)], idx_slab)
    pltpu.sync_copy(val_hbm.at[pl.ds(base * LP, RPW * LP)], val_slab)

    bufs = (buf0, buf1)
    sems = (sem0, sem1)

    for b in range(NBUF):
        pltpu.async_copy(
            emb_hbm.at[idx_slab.at[pl.ds(b * CR * LP, CR * LP)]],
            bufs[b], sems[b])

    def step(c, carry):
        for b in range(NBUF):
            chunk = c * NBUF + b
            pltpu.make_async_copy(
                emb_hbm.at[idx_slab.at[pl.ds(chunk * CR * LP, CR * LP)]],
                bufs[b], sems[b]).wait()

            def row_step(r, carry2):
                row = chunk * CR + r
                accs = [jnp.zeros((16,), jnp.float32) for _ in range(4)]
                for l in range(LP):
                    wv = plsc.load_gather(
                        val_slab,
                        [jnp.full((16,), row * LP + l, jnp.int32)])
                    for k in range(4):
                        accs[k] = accs[k] + (
                            bufs[b][r * LP + l, pl.ds(k * 16, 16)] * wv)
                for k in range(4):
                    out_v[row, pl.ds(k * 16, 16)] = accs[k]
                return carry2

            lax.fori_loop(0, CR, row_step, 0)
            nxt = chunk + NBUF

            @pl.when(nxt < NCH)
            def _():
                pltpu.async_copy(
                    emb_hbm.at[idx_slab.at[pl.ds(nxt * CR * LP, CR * LP)]],
                    bufs[b], sems[b])
        return carry

    lax.fori_loop(0, NCH // NBUF, step, 0)

    pltpu.sync_copy(out_v, out_hbm.at[pl.ds(base, RPW), :])


@functools.partial(jax.jit, static_argnames=())
def _lookup(idx_flat, val_flat, embedding):
    mesh = plsc.VectorSubcoreMesh(core_axis_name="c", subcore_axis_name="s")
    return pl.kernel(
        _body,
        out_type=jax.ShapeDtypeStruct((B, D), jnp.float32),
        mesh=mesh,
        compiler_params=pltpu.CompilerParams(
            needs_layout_passes=False, use_tc_tiling_on_sc=False),
        scratch_types=[
            pltpu.VMEM((RPW * LP,), jnp.int32),
            pltpu.VMEM((RPW * LP,), jnp.float32),
            pltpu.VMEM((RPW, D), jnp.float32),
            pltpu.VMEM((CR * LP, D), jnp.float32),
            pltpu.VMEM((CR * LP, D), jnp.float32),
            pltpu.SemaphoreType.DMA,
            pltpu.SemaphoreType.DMA,
        ],
    )(idx_flat, val_flat, embedding)


def kernel(idx, val, embedding):
    idx_p = jnp.pad(idx.astype(jnp.int32), ((0, 0), (0, LP - L)))
    val_p = jnp.pad(val.astype(jnp.float32), ((0, 0), (0, LP - L)))
    out = _lookup(idx_p.reshape(-1), val_p.reshape(-1), embedding)
    return out[:, None, :]
